# initial kernel scaffold (unmeasured)
import jax
import jax.numpy as jnp
from jax import lax
from jax.experimental import pallas as pl
from jax.experimental.pallas import tpu as pltpu

N_DEV = 4


def _allgather_kv(K, V):
    B, S, H, D = K.shape

    def body(k_ref, v_ref, kfull_ref, vfull_ref, k_send, k_recv, v_send, v_recv):
        my = lax.axis_index("i")
        left = lax.rem(my + N_DEV - 1, N_DEV)
        right = lax.rem(my + 1, N_DEV)

        barrier = pltpu.get_barrier_semaphore()
        pl.semaphore_signal(
            barrier, inc=1, device_id=(left,), device_id_type=pl.DeviceIdType.MESH
        )
        pl.semaphore_signal(
            barrier, inc=1, device_id=(right,), device_id_type=pl.DeviceIdType.MESH
        )
        pl.semaphore_wait(barrier, 2)

        kfull_ref[:, pl.ds(my * S, S), :, :] = k_ref[...].astype(jnp.bfloat16)
        vfull_ref[:, pl.ds(my * S, S), :, :] = v_ref[...].astype(jnp.bfloat16)

        for h in range(N_DEV - 1):
            ko = lax.rem(my + N_DEV - h, N_DEV)
            vo = lax.rem(my + h, N_DEV)
            k_rdma = pltpu.make_async_remote_copy(
                src_ref=kfull_ref.at[:, pl.ds(ko * S, S), :, :],
                dst_ref=kfull_ref.at[:, pl.ds(ko * S, S), :, :],
                send_sem=k_send.at[h],
                recv_sem=k_recv.at[h],
                device_id=(right,),
                device_id_type=pl.DeviceIdType.MESH,
            )
            v_rdma = pltpu.make_async_remote_copy(
                src_ref=vfull_ref.at[:, pl.ds(vo * S, S), :, :],
                dst_ref=vfull_ref.at[:, pl.ds(vo * S, S), :, :],
                send_sem=v_send.at[h],
                recv_sem=v_recv.at[h],
                device_id=(left,),
                device_id_type=pl.DeviceIdType.MESH,
            )
            k_rdma.start()
            v_rdma.start()
            k_rdma.wait()
            v_rdma.wait()

    return pl.pallas_call(
        body,
        out_shape=[
            jax.ShapeDtypeStruct((B, N_DEV * S, H, D), jnp.bfloat16),
            jax.ShapeDtypeStruct((B, N_DEV * S, H, D), jnp.bfloat16),
        ],
        in_specs=[
            pl.BlockSpec(memory_space=pltpu.VMEM),
            pl.BlockSpec(memory_space=pltpu.VMEM),
        ],
        out_specs=[
            pl.BlockSpec(memory_space=pltpu.VMEM),
            pl.BlockSpec(memory_space=pltpu.VMEM),
        ],
        scratch_shapes=[
            pltpu.SemaphoreType.DMA((N_DEV - 1,)),
            pltpu.SemaphoreType.DMA((N_DEV - 1,)),
            pltpu.SemaphoreType.DMA((N_DEV - 1,)),
            pltpu.SemaphoreType.DMA((N_DEV - 1,)),
        ],
        compiler_params=pltpu.CompilerParams(collective_id=0),
    )(K, V)


def _attention(Q, Kfull, Vfull):
    B, S, H, D = Q.shape
    SK = Kfull.shape[1]
    scale = D**-0.5

    def body(q_ref, k_ref, v_ref, o_ref):
        q = q_ref[0, :, 0, :].astype(jnp.bfloat16)
        k = k_ref[0, :, 0, :]
        v = v_ref[0, :, 0, :]
        s = lax.dot_general(
            q, k, (((1,), (1,)), ((), ())), preferred_element_type=jnp.float32
        )
        s = s * scale
        m = jnp.max(s, axis=-1, keepdims=True)
        p = jnp.exp(s - m)
        p = p / jnp.sum(p, axis=-1, keepdims=True)
        o = lax.dot_general(
            p.astype(jnp.bfloat16),
            v,
            (((1,), (0,)), ((), ())),
            preferred_element_type=jnp.float32,
        )
        o_ref[0, :, 0, :] = o

    return pl.pallas_call(
        body,
        grid=(B, H),
        out_shape=jax.ShapeDtypeStruct((B, S, H, D), jnp.float32),
        in_specs=[
            pl.BlockSpec((1, S, 1, D), lambda b, h: (b, 0, h, 0)),
            pl.BlockSpec((1, SK, 1, D), lambda b, h: (b, 0, h, 0)),
            pl.BlockSpec((1, SK, 1, D), lambda b, h: (b, 0, h, 0)),
        ],
        out_specs=pl.BlockSpec((1, S, 1, D), lambda b, h: (b, 0, h, 0)),
    )(Q, Kfull, Vfull)


def kernel(Q, K, V):
    Kfull, Vfull = _allgather_kv(K, V)
    return _attention(Q, Kfull, Vfull)


# baseline (device time: 283796 ns/iter reference)
import jax
import jax.numpy as jnp
from jax import lax
from jax.experimental import pallas as pl
from jax.experimental.pallas import tpu as pltpu

N_DEV = 4


def _allgather_kv(K, V):
    B, S, H, D = K.shape

    def body(k_ref, v_ref, kfull_ref, vfull_ref, k_send, k_recv, v_send, v_recv):
        my = lax.axis_index("i")
        left = lax.rem(my + N_DEV - 1, N_DEV)
        right = lax.rem(my + 1, N_DEV)

        barrier = pltpu.get_barrier_semaphore()
        pl.semaphore_signal(
            barrier, inc=1, device_id=(left,), device_id_type=pl.DeviceIdType.MESH
        )
        pl.semaphore_signal(
            barrier, inc=1, device_id=(right,), device_id_type=pl.DeviceIdType.MESH
        )
        pl.semaphore_wait(barrier, 2)

        kfull_ref[:, pl.ds(my * S, S), :, :] = k_ref[...].astype(jnp.bfloat16)
        vfull_ref[:, pl.ds(my * S, S), :, :] = v_ref[...].astype(jnp.bfloat16)

        for h in range(N_DEV - 1):
            ko = lax.rem(my + N_DEV - h, N_DEV)
            vo = lax.rem(my + h, N_DEV)
            k_rdma = pltpu.make_async_remote_copy(
                src_ref=kfull_ref.at[:, pl.ds(ko * S, S), :, :],
                dst_ref=kfull_ref.at[:, pl.ds(ko * S, S), :, :],
                send_sem=k_send.at[h],
                recv_sem=k_recv.at[h],
                device_id=(right,),
                device_id_type=pl.DeviceIdType.MESH,
            )
            v_rdma = pltpu.make_async_remote_copy(
                src_ref=vfull_ref.at[:, pl.ds(vo * S, S), :, :],
                dst_ref=vfull_ref.at[:, pl.ds(vo * S, S), :, :],
                send_sem=v_send.at[h],
                recv_sem=v_recv.at[h],
                device_id=(left,),
                device_id_type=pl.DeviceIdType.MESH,
            )
            k_rdma.start()
            v_rdma.start()
            k_rdma.wait()
            v_rdma.wait()

    return pl.pallas_call(
        body,
        out_shape=[
            jax.ShapeDtypeStruct((B, N_DEV * S, H, D), jnp.bfloat16),
            jax.ShapeDtypeStruct((B, N_DEV * S, H, D), jnp.bfloat16),
        ],
        in_specs=[
            pl.BlockSpec(memory_space=pltpu.VMEM),
            pl.BlockSpec(memory_space=pltpu.VMEM),
        ],
        out_specs=[
            pl.BlockSpec(memory_space=pltpu.VMEM),
            pl.BlockSpec(memory_space=pltpu.VMEM),
        ],
        scratch_shapes=[
            pltpu.SemaphoreType.DMA((N_DEV - 1,)),
            pltpu.SemaphoreType.DMA((N_DEV - 1,)),
            pltpu.SemaphoreType.DMA((N_DEV - 1,)),
            pltpu.SemaphoreType.DMA((N_DEV - 1,)),
        ],
        compiler_params=pltpu.CompilerParams(collective_id=0),
    )(K, V)


def _attention(Q, Kfull, Vfull):
    B, S, H, D = Q.shape
    SK = Kfull.shape[1]
    scale = D**-0.5

    def body(q_ref, k_ref, v_ref, o_ref):
        for h in range(H):
            q = q_ref[0, :, h, :].astype(jnp.bfloat16)
            k = k_ref[0, :, h, :]
            v = v_ref[0, :, h, :]
            s = lax.dot_general(
                q, k, (((1,), (1,)), ((), ())), preferred_element_type=jnp.float32
            )
            s = s * scale
            m = jnp.max(s, axis=-1, keepdims=True)
            p = jnp.exp(s - m)
            p = p / jnp.sum(p, axis=-1, keepdims=True)
            o = lax.dot_general(
                p.astype(jnp.bfloat16),
                v,
                (((1,), (0,)), ((), ())),
                preferred_element_type=jnp.float32,
            )
            o_ref[0, :, h, :] = o

    return pl.pallas_call(
        body,
        grid=(B,),
        out_shape=jax.ShapeDtypeStruct((B, S, H, D), jnp.float32),
        in_specs=[
            pl.BlockSpec((1, S, H, D), lambda b: (b, 0, 0, 0)),
            pl.BlockSpec((1, SK, H, D), lambda b: (b, 0, 0, 0)),
            pl.BlockSpec((1, SK, H, D), lambda b: (b, 0, 0, 0)),
        ],
        out_specs=pl.BlockSpec((1, S, H, D), lambda b: (b, 0, 0, 0)),
    )(Q, Kfull, Vfull)


def kernel(Q, K, V):
    Kfull, Vfull = _allgather_kv(K, V)
    return _attention(Q, Kfull, Vfull)


# device time: 138044 ns/iter; 2.0558x vs baseline; 2.0558x over previous
import jax
import jax.numpy as jnp
from jax import lax
from jax.experimental import pallas as pl
from jax.experimental.pallas import tpu as pltpu

N_DEV = 4


def _allgather_kv(K, V):
    B, S, HD = K.shape

    def body(k_ref, v_ref, kfull_ref, vfull_ref, k_send, k_recv, v_send, v_recv):
        my = lax.axis_index("i")
        left = lax.rem(my + N_DEV - 1, N_DEV)
        right = lax.rem(my + 1, N_DEV)

        barrier = pltpu.get_barrier_semaphore()
        pl.semaphore_signal(
            barrier, inc=1, device_id=(left,), device_id_type=pl.DeviceIdType.MESH
        )
        pl.semaphore_signal(
            barrier, inc=1, device_id=(right,), device_id_type=pl.DeviceIdType.MESH
        )
        pl.semaphore_wait(barrier, 2)

        kfull_ref[:, pl.ds(my * S, S), :] = k_ref[...].astype(jnp.bfloat16)
        vfull_ref[:, pl.ds(my * S, S), :] = v_ref[...].astype(jnp.bfloat16)

        for h in range(N_DEV - 1):
            ko = lax.rem(my + N_DEV - h, N_DEV)
            vo = lax.rem(my + h, N_DEV)
            k_rdma = pltpu.make_async_remote_copy(
                src_ref=kfull_ref.at[:, pl.ds(ko * S, S), :],
                dst_ref=kfull_ref.at[:, pl.ds(ko * S, S), :],
                send_sem=k_send.at[h],
                recv_sem=k_recv.at[h],
                device_id=(right,),
                device_id_type=pl.DeviceIdType.MESH,
            )
            v_rdma = pltpu.make_async_remote_copy(
                src_ref=vfull_ref.at[:, pl.ds(vo * S, S), :],
                dst_ref=vfull_ref.at[:, pl.ds(vo * S, S), :],
                send_sem=v_send.at[h],
                recv_sem=v_recv.at[h],
                device_id=(left,),
                device_id_type=pl.DeviceIdType.MESH,
            )
            k_rdma.start()
            v_rdma.start()
            k_rdma.wait()
            v_rdma.wait()

    return pl.pallas_call(
        body,
        out_shape=[
            jax.ShapeDtypeStruct((B, N_DEV * S, HD), jnp.bfloat16),
            jax.ShapeDtypeStruct((B, N_DEV * S, HD), jnp.bfloat16),
        ],
        in_specs=[
            pl.BlockSpec(memory_space=pltpu.VMEM),
            pl.BlockSpec(memory_space=pltpu.VMEM),
        ],
        out_specs=[
            pl.BlockSpec(memory_space=pltpu.VMEM),
            pl.BlockSpec(memory_space=pltpu.VMEM),
        ],
        scratch_shapes=[
            pltpu.SemaphoreType.DMA((N_DEV - 1,)),
            pltpu.SemaphoreType.DMA((N_DEV - 1,)),
            pltpu.SemaphoreType.DMA((N_DEV - 1,)),
            pltpu.SemaphoreType.DMA((N_DEV - 1,)),
        ],
        compiler_params=pltpu.CompilerParams(collective_id=0),
    )(K, V)


def _attention(Q, Kfull, Vfull, H, D):
    B, S, HD = Q.shape
    SK = Kfull.shape[1]
    scale = D**-0.5

    def body(q_ref, k_ref, v_ref, o_ref):
        qb = q_ref[0].astype(jnp.bfloat16)
        kb = k_ref[0]
        vb = v_ref[0]
        for h in range(H):
            q = qb[:, h * D : (h + 1) * D]
            k = kb[:, h * D : (h + 1) * D]
            v = vb[:, h * D : (h + 1) * D]
            s = lax.dot_general(
                q, k, (((1,), (1,)), ((), ())), preferred_element_type=jnp.float32
            )
            s = s * scale
            m = jnp.max(s, axis=-1, keepdims=True)
            p = jnp.exp(s - m)
            p = p / jnp.sum(p, axis=-1, keepdims=True)
            o = lax.dot_general(
                p.astype(jnp.bfloat16),
                v,
                (((1,), (0,)), ((), ())),
                preferred_element_type=jnp.float32,
            )
            o_ref[0, :, h * D : (h + 1) * D] = o

    return pl.pallas_call(
        body,
        grid=(B,),
        out_shape=jax.ShapeDtypeStruct((B, S, HD), jnp.float32),
        in_specs=[
            pl.BlockSpec((1, S, HD), lambda b: (b, 0, 0)),
            pl.BlockSpec((1, SK, HD), lambda b: (b, 0, 0)),
            pl.BlockSpec((1, SK, HD), lambda b: (b, 0, 0)),
        ],
        out_specs=pl.BlockSpec((1, S, HD), lambda b: (b, 0, 0)),
    )(Q, Kfull, Vfull)


def kernel(Q, K, V):
    B, S, H, D = Q.shape
    Q3 = Q.reshape(B, S, H * D)
    K3 = K.reshape(B, S, H * D)
    V3 = V.reshape(B, S, H * D)
    Kfull, Vfull = _allgather_kv(K3, V3)
    out = _attention(Q3, Kfull, Vfull, H, D)
    return out.reshape(B, S, H, D)


# device time: 122063 ns/iter; 2.3250x vs baseline; 1.1309x over previous
import jax
import jax.numpy as jnp
from jax import lax
from jax.experimental import pallas as pl
from jax.experimental.pallas import tpu as pltpu

N_DEV = 4
LOG2E = 1.4426950408889634


def _fused_ag_attention(Q, K, V, H, D):
    B, S, HD = Q.shape
    HALF = HD // 2
    HGRP = H // 2
    scale = D**-0.5

    def body(
        q_ref, k_ref, v_ref, o_ref, kbuf, vbuf, acc, lbuf,
        sa_k, sa_v, sb_k, sb_v, ra_k, ra_v, rb_k, rb_v,
    ):
        h = pl.program_id(0)
        b = pl.program_id(1)
        my = lax.axis_index("i")
        left = lax.rem(my + N_DEV - 1, N_DEV)
        right = lax.rem(my + 1, N_DEV)
        barrier = pltpu.get_barrier_semaphore()

        def rc(buf, o, half, send_sem, recv_sem, dev, hc):
            sl = pl.ds(half * HALF, HALF)
            return pltpu.make_async_remote_copy(
                src_ref=buf.at[o, :, :, sl],
                dst_ref=buf.at[o, :, :, sl],
                send_sem=send_sem.at[hc],
                recv_sem=recv_sem.at[hc],
                device_id=(dev,),
                device_id_type=pl.DeviceIdType.MESH,
            )

        def mk_sends(hc):
            oA = lax.rem(my + N_DEV - hc, N_DEV)
            oB = lax.rem(my + hc, N_DEV)
            return [
                rc(kbuf, oA, 0, sa_k, ra_k, right, hc),
                rc(vbuf, oA, 0, sa_v, ra_v, right, hc),
                rc(kbuf, oB, 1, sb_k, rb_k, left, hc),
                rc(vbuf, oB, 1, sb_v, rb_v, left, hc),
            ]

        def mk_recvs(hc):
            oA = lax.rem(my + N_DEV - hc - 1, N_DEV)
            oB = lax.rem(my + hc + 1, N_DEV)
            return [
                rc(kbuf, oA, 0, sa_k, ra_k, left, hc),
                rc(vbuf, oA, 0, sa_v, ra_v, left, hc),
                rc(kbuf, oB, 1, sb_k, rb_k, right, hc),
                rc(vbuf, oB, 1, sb_v, rb_v, right, hc),
            ]

        @pl.when(jnp.logical_and(h == 0, b == 0))
        def _():
            pl.semaphore_signal(
                barrier, inc=1, device_id=(left,),
                device_id_type=pl.DeviceIdType.MESH,
            )
            pl.semaphore_signal(
                barrier, inc=1, device_id=(right,),
                device_id_type=pl.DeviceIdType.MESH,
            )
            pl.semaphore_wait(barrier, 2)
            kbuf[my] = k_ref[...].astype(jnp.bfloat16)
            vbuf[my] = v_ref[...].astype(jnp.bfloat16)
            for d in mk_sends(0):
                d.start()

        for hc in (1, 2, 3):
            @pl.when(jnp.logical_and(h == hc, b == 0))
            def _(hc=hc):
                for d in mk_sends(hc - 1):
                    d.wait_send()
                for d in mk_recvs(hc - 1):
                    d.wait_recv()
                if hc < N_DEV - 1:
                    for d in mk_sends(hc):
                        d.start()

        oA = lax.rem(my + N_DEV - h, N_DEV)
        oB = lax.rem(my + h, N_DEV)
        is_first = h == 0
        q = (q_ref[b] * (scale * LOG2E)).astype(jnp.bfloat16)
        kA = kbuf[oA, b, :, 0:HALF]
        vA = vbuf[oA, b, :, 0:HALF]
        kB = kbuf[oB, b, :, HALF:HD]
        vB = vbuf[oB, b, :, HALF:HD]
        for hi in range(H):
            if hi < HGRP:
                kh = kA[:, hi * D : (hi + 1) * D]
                vh = vA[:, hi * D : (hi + 1) * D]
            else:
                kh = kB[:, (hi - HGRP) * D : (hi - HGRP + 1) * D]
                vh = vB[:, (hi - HGRP) * D : (hi - HGRP + 1) * D]
            qh = q[:, hi * D : (hi + 1) * D]
            s = lax.dot_general(
                qh, kh, (((1,), (1,)), ((), ())),
                preferred_element_type=jnp.float32,
            )
            p = jnp.exp2(s)
            lh = jnp.sum(p, axis=-1, keepdims=True)
            oc = lax.dot_general(
                p.astype(jnp.bfloat16), vh, (((1,), (0,)), ((), ())),
                preferred_element_type=jnp.float32,
            )
            a_prev = jnp.where(is_first, 0.0, acc[b, :, hi * D : (hi + 1) * D])
            l_prev = jnp.where(is_first, 0.0, lbuf[b, :, hi : hi + 1])
            a_new = a_prev + oc
            l_new = l_prev + lh
            acc[b, :, hi * D : (hi + 1) * D] = a_new
            lbuf[b, :, hi : hi + 1] = l_new
            o_ref[0, :, hi * D : (hi + 1) * D] = a_new * (1.0 / l_new)

    return pl.pallas_call(
        body,
        grid=(N_DEV, B),
        out_shape=jax.ShapeDtypeStruct((B, S, HD), jnp.float32),
        in_specs=[
            pl.BlockSpec(memory_space=pltpu.VMEM),
            pl.BlockSpec(memory_space=pltpu.VMEM),
            pl.BlockSpec(memory_space=pltpu.VMEM),
        ],
        out_specs=pl.BlockSpec((1, S, HD), lambda h, b: (b, 0, 0)),
        scratch_shapes=[
            pltpu.VMEM((N_DEV, B, S, HD), jnp.bfloat16),
            pltpu.VMEM((N_DEV, B, S, HD), jnp.bfloat16),
            pltpu.VMEM((B, S, HD), jnp.float32),
            pltpu.VMEM((B, S, 128), jnp.float32),
        ]
        + [pltpu.SemaphoreType.DMA((N_DEV - 1,)) for _ in range(8)],
        compiler_params=pltpu.CompilerParams(
            collective_id=0,
            dimension_semantics=("arbitrary", "arbitrary"),
        ),
    )(Q, K, V)


def kernel(Q, K, V):
    B, S, H, D = Q.shape
    out = _fused_ag_attention(
        Q.reshape(B, S, H * D), K.reshape(B, S, H * D), V.reshape(B, S, H * D), H, D
    )
    return out.reshape(B, S, H, D)


# device time: 96710 ns/iter; 2.9345x vs baseline; 1.2622x over previous
import jax
import jax.numpy as jnp
from jax import lax
from jax.experimental import pallas as pl
from jax.experimental.pallas import tpu as pltpu

N_DEV = 4
LOG2E = 1.4426950408889634


def _fused_ag_attention(Q, K, V, H, D):
    B, S, HD = Q.shape
    HALF = HD // 2
    HGRP = H // 2

    def body(
        q_ref, k_ref, v_ref, o_ref, kbuf, vbuf, acc, lbuf,
        sa_k, sa_v, sb_k, sb_v, ra_k, ra_v, rb_k, rb_v,
    ):
        h = pl.program_id(0)
        my = lax.axis_index("i")
        left = lax.rem(my + N_DEV - 1, N_DEV)
        right = lax.rem(my + 1, N_DEV)
        barrier = pltpu.get_barrier_semaphore()

        SP = S // 2

        def rc(src, dst_o, half, send_sem, recv_sem, dev, hc, pc):
            sl = pl.ds(half * HALF, HALF)
            sq = pl.ds(pc * SP, SP)
            return pltpu.make_async_remote_copy(
                src_ref=src,
                dst_ref=(kbuf if send_sem is sa_k or send_sem is sb_k else vbuf)
                .at[dst_o, :, sq, sl],
                send_sem=send_sem.at[hc, pc],
                recv_sem=recv_sem.at[hc, pc],
                device_id=(dev,),
                device_id_type=pl.DeviceIdType.MESH,
            )

        def buf_slice(buf, o, half, pc):
            return buf.at[o, :, pl.ds(pc * SP, SP), pl.ds(half * HALF, HALF)]

        def in_slice(ref, half, pc):
            return ref.at[:, pl.ds(pc * SP, SP), pl.ds(half * HALF, HALF)]

        def mk_sends(hc, pc):
            oA = lax.rem(my + N_DEV - hc, N_DEV)
            oB = lax.rem(my + hc, N_DEV)
            if hc == 0:
                srcs = [
                    in_slice(k_ref, 0, pc), in_slice(v_ref, 0, pc),
                    in_slice(k_ref, 1, pc), in_slice(v_ref, 1, pc),
                ]
            else:
                srcs = [
                    buf_slice(kbuf, oA, 0, pc), buf_slice(vbuf, oA, 0, pc),
                    buf_slice(kbuf, oB, 1, pc), buf_slice(vbuf, oB, 1, pc),
                ]
            return [
                rc(srcs[0], oA, 0, sa_k, ra_k, right, hc, pc),
                rc(srcs[1], oA, 0, sa_v, ra_v, right, hc, pc),
                rc(srcs[2], oB, 1, sb_k, rb_k, left, hc, pc),
                rc(srcs[3], oB, 1, sb_v, rb_v, left, hc, pc),
            ]

        def mk_recvs(hc, pc):
            oA = lax.rem(my + N_DEV - hc - 1, N_DEV)
            oB = lax.rem(my + hc + 1, N_DEV)
            return [
                rc(buf_slice(kbuf, oA, 0, pc), oA, 0, sa_k, ra_k, left, hc, pc),
                rc(buf_slice(vbuf, oA, 0, pc), oA, 0, sa_v, ra_v, left, hc, pc),
                rc(buf_slice(kbuf, oB, 1, pc), oB, 1, sb_k, rb_k, right, hc, pc),
                rc(buf_slice(vbuf, oB, 1, pc), oB, 1, sb_v, rb_v, right, hc, pc),
            ]

        @pl.when(h == 0)
        def _():
            pl.semaphore_signal(
                barrier, inc=1, device_id=(left,),
                device_id_type=pl.DeviceIdType.MESH,
            )
            pl.semaphore_signal(
                barrier, inc=1, device_id=(right,),
                device_id_type=pl.DeviceIdType.MESH,
            )
            pl.semaphore_wait(barrier, 2)
            for pc in (0, 1):
                for d in mk_sends(0, pc):
                    d.start()
            kbuf[my] = k_ref[...]
            vbuf[my] = v_ref[...]

        for hc in (1, 2, 3):
            @pl.when(h == hc)
            def _(hc=hc):
                for pc in (0, 1):
                    for d in mk_recvs(hc - 1, pc):
                        d.wait_recv()
                    if hc < N_DEV - 1:
                        for d in mk_sends(hc, pc):
                            d.start()
                for pc in (0, 1):
                    for d in mk_sends(hc - 1, pc):
                        d.wait_send()

        oA = lax.rem(my + N_DEV - h, N_DEV)
        oB = lax.rem(my + h, N_DEV)
        is_first = h == 0
        is_last = h == N_DEV - 1

        for b in range(B):
            q = q_ref[b]
            kA = kbuf[oA, b, :, 0:HALF]
            vA = vbuf[oA, b, :, 0:HALF]
            kB = kbuf[oB, b, :, HALF:HD]
            vB = vbuf[oB, b, :, HALF:HD]

            def head_chunk(hi, kA=kA, vA=vA, kB=kB, vB=vB, q=q):
                if hi < HGRP:
                    kh = kA[:, hi * D : (hi + 1) * D]
                    vh = vA[:, hi * D : (hi + 1) * D]
                else:
                    kh = kB[:, (hi - HGRP) * D : (hi - HGRP + 1) * D]
                    vh = vB[:, (hi - HGRP) * D : (hi - HGRP + 1) * D]
                qh = q[:, hi * D : (hi + 1) * D]
                s = lax.dot_general(
                    qh, kh, (((1,), (1,)), ((), ())),
                    preferred_element_type=jnp.float32,
                )
                p = jnp.exp2(s)
                lh = jnp.sum(p, axis=-1, keepdims=True)
                oc = lax.dot_general(
                    p.astype(jnp.bfloat16), vh, (((1,), (0,)), ((), ())),
                    preferred_element_type=jnp.float32,
                )
                return oc, lh

            a_pairs = []
            l_cols = []
            for j in range(H // 2):
                oc0, lh0 = head_chunk(2 * j)
                oc1, lh1 = head_chunk(2 * j + 1)
                oc = jnp.concatenate([oc0, oc1], axis=1)
                a_prev = jnp.where(
                    is_first, 0.0, acc[b, :, j * 2 * D : (j + 1) * 2 * D]
                )
                a_new = a_prev + oc
                acc[b, :, j * 2 * D : (j + 1) * 2 * D] = a_new
                a_pairs.append(a_new)
                l_cols.extend([lh0, lh1])

            l_step = jnp.concatenate(l_cols, axis=1)
            l_prev = jnp.where(is_first, 0.0, lbuf[b, :, 0:H])
            l_new = l_prev + l_step
            lbuf[b, :, 0:H] = l_new

            @pl.when(is_last)
            def _(b=b, a_pairs=a_pairs, l_new=l_new):
                for j in range(H // 2):
                    r0 = 1.0 / l_new[:, 2 * j : 2 * j + 1]
                    r1 = 1.0 / l_new[:, 2 * j + 1 : 2 * j + 2]
                    r = jnp.concatenate(
                        [jnp.broadcast_to(r0, (S, D)), jnp.broadcast_to(r1, (S, D))],
                        axis=1,
                    )
                    o_ref[b, :, j * 2 * D : (j + 1) * 2 * D] = (
                        a_pairs[j] * r
                    ).astype(jnp.bfloat16)

    return pl.pallas_call(
        body,
        grid=(N_DEV,),
        out_shape=jax.ShapeDtypeStruct((B, S, HD), jnp.bfloat16),
        in_specs=[
            pl.BlockSpec(memory_space=pltpu.VMEM),
            pl.BlockSpec(memory_space=pltpu.VMEM),
            pl.BlockSpec(memory_space=pltpu.VMEM),
        ],
        out_specs=pl.BlockSpec(memory_space=pltpu.VMEM),
        scratch_shapes=[
            pltpu.VMEM((N_DEV, B, S, HD), jnp.bfloat16),
            pltpu.VMEM((N_DEV, B, S, HD), jnp.bfloat16),
            pltpu.VMEM((B, S, HD), jnp.float32),
            pltpu.VMEM((B, S, 128), jnp.float32),
        ]
        + [pltpu.SemaphoreType.DMA((N_DEV - 1, 2)) for _ in range(8)],
        compiler_params=pltpu.CompilerParams(
            collective_id=0,
            dimension_semantics=("arbitrary",),
        ),
    )(Q, K, V)


def kernel(Q, K, V):
    B, S, H, D = Q.shape
    scale = D**-0.5
    Qs = (Q * (scale * LOG2E)).astype(jnp.bfloat16).reshape(B, S, H * D)
    K3 = K.astype(jnp.bfloat16).reshape(B, S, H * D)
    V3 = V.astype(jnp.bfloat16).reshape(B, S, H * D)
    out = _fused_ag_attention(Qs, K3, V3, H, D)
    return out.reshape(B, S, H, D)
